# speedup vs baseline: 1.7807x; 1.7807x over previous
import functools

import jax
import jax.numpy as jnp
from jax.experimental import pallas as pl

NUM_NODES = 2048
IN_DIM = 128
HIDDEN_DIM = 128
OUT_DIM = 128
HEADS = 4
BATCH = 2
JB = 256


def _headmat(a):
    H, F = a.shape
    eye = jnp.eye(H, dtype=a.dtype)
    return (eye[:, :, None] * a[:, None, :]).reshape(H, H * F)


def _dot_t(a, b):
    return jax.lax.dot_general(a, b, (((1,), (1,)), ((), ())),
                               preferred_element_type=jnp.float32)


def _prep1_body(emb_ref, x_ref, pW_ref, pb_ref, W1_ref, As_ref, Ad_ref,
                mask_ref, wh_ref, es_ref, ed_ref):
    emb = emb_ref[...]
    ne = emb / (jnp.sqrt(jnp.sum(emb * emb, axis=1, keepdims=True)) + 1e-12)
    adj = _dot_t(ne, ne)
    mask_ref[...] = (adj > 0.5).astype(jnp.float32)
    x = x_ref[...].reshape(BATCH * NUM_NODES, IN_DIM)
    h = jnp.dot(x, pW_ref[...], preferred_element_type=jnp.float32) + pb_ref[...][None, :]
    wh = jnp.dot(h, W1_ref[...], preferred_element_type=jnp.float32)
    wh3 = wh.reshape(BATCH, NUM_NODES, HIDDEN_DIM)
    wh_ref[...] = wh3
    for b in range(BATCH):
        es_ref[b] = _dot_t(As_ref[...], wh3[b])
        ed_ref[b] = _dot_t(Ad_ref[...], wh3[b])


def _prep2_body(h_ref, W_ref, As_ref, Ad_ref, wh_ref, es_ref, ed_ref):
    h = h_ref[...].reshape(BATCH * NUM_NODES, HIDDEN_DIM)
    wh = jnp.dot(h, W_ref[...], preferred_element_type=jnp.float32)
    wh3 = wh.reshape(BATCH, NUM_NODES, HIDDEN_DIM)
    wh_ref[...] = wh3
    for b in range(BATCH):
        es_ref[b] = _dot_t(As_ref[...], wh3[b])
        ed_ref[b] = _dot_t(Ad_ref[...], wh3[b])


def _attn_body(mask_ref, wh_ref, es_ref, ed_ref, bias_ref, out_ref, *, relu):
    F = HIDDEN_DIM // HEADS
    maskb = mask_ref[...]
    for b in range(BATCH):
        whb = wh_ref[b]
        outs = []
        for h in range(HEADS):
            s = es_ref[b, h, :]
            d = ed_ref[b, h, :]
            z = s[None, :] + d[:, None]
            z = jnp.where(z >= 0, z, 0.2 * z)
            z = jnp.where(maskb > 0, z, -1e9)
            m = jnp.max(z, axis=1, keepdims=True)
            p = jnp.exp(z - m)
            alpha = p / jnp.sum(p, axis=1, keepdims=True)
            outs.append(jnp.dot(alpha, whb[:, h * F:(h + 1) * F],
                                preferred_element_type=jnp.float32))
        o = jnp.concatenate(outs, axis=1) + bias_ref[...][None, :]
        if relu:
            o = jnp.maximum(o, 0.0)
        out_ref[b] = o


def _attn_layer(mask, wh, es, ed, bias, relu):
    N, HD = NUM_NODES, HIDDEN_DIM
    grid = (N // JB,)
    return pl.pallas_call(
        functools.partial(_attn_body, relu=relu),
        grid=grid,
        in_specs=[
            pl.BlockSpec((JB, N), lambda j: (j, 0)),
            pl.BlockSpec((BATCH, N, HD), lambda j: (0, 0, 0)),
            pl.BlockSpec((BATCH, HEADS, N), lambda j: (0, 0, 0)),
            pl.BlockSpec((BATCH, HEADS, JB), lambda j: (0, 0, j)),
            pl.BlockSpec((HD,), lambda j: (0,)),
        ],
        out_specs=pl.BlockSpec((BATCH, JB, HD), lambda j: (0, j, 0)),
        out_shape=jax.ShapeDtypeStruct((BATCH, N, HD), jnp.float32),
    )(mask, wh, es, ed, bias)


def kernel(x, embedding, proj_W, proj_b, W1, a1_src, a1_dst, b1,
           W2, a2_src, a2_dst, b2):
    N, HD = NUM_NODES, HIDDEN_DIM
    A1s, A1d = _headmat(a1_src), _headmat(a1_dst)
    A2s, A2d = _headmat(a2_src), _headmat(a2_dst)

    mask, wh1, e1s, e1d = pl.pallas_call(
        _prep1_body,
        out_shape=(
            jax.ShapeDtypeStruct((N, N), jnp.float32),
            jax.ShapeDtypeStruct((BATCH, N, HD), jnp.float32),
            jax.ShapeDtypeStruct((BATCH, HEADS, N), jnp.float32),
            jax.ShapeDtypeStruct((BATCH, HEADS, N), jnp.float32),
        ),
    )(embedding, x, proj_W, proj_b, W1, A1s, A1d)

    h1 = _attn_layer(mask, wh1, e1s, e1d, b1, relu=True)

    wh2, e2s, e2d = pl.pallas_call(
        _prep2_body,
        out_shape=(
            jax.ShapeDtypeStruct((BATCH, N, HD), jnp.float32),
            jax.ShapeDtypeStruct((BATCH, HEADS, N), jnp.float32),
            jax.ShapeDtypeStruct((BATCH, HEADS, N), jnp.float32),
        ),
    )(h1, W2, A2s, A2d)

    return _attn_layer(mask, wh2, e2s, e2d, b2, relu=False)
